# baseline (device time: 205010 ns/iter reference)
import jax
import jax.numpy as jnp
from jax import lax
from jax.experimental import pallas as pl
from jax.experimental.pallas import tpu as pltpu

N_Z = 4
H, Dh, Dr = 16, 128, 32
DC = 128
SCALE = (Dh + Dr) ** -0.5
F32 = jnp.float32


def _matmul(a, b):
    m, _ = a.shape
    _, n = b.shape

    def body(a_ref, b_ref, o_ref):
        o_ref[...] = jnp.dot(a_ref[...], b_ref[...], preferred_element_type=F32)

    return pl.pallas_call(
        body, out_shape=jax.ShapeDtypeStruct((m, n), F32),
        compiler_params=pltpu.CompilerParams(
            vmem_limit_bytes=63 * 1024 * 1024),
    )(a, b)


def _proj_lite(x2, Wdkv, Wqr, Wkr):
    m = x2.shape[0]

    def body(x_ref, wdkv_ref, wqr_ref, wkr_ref, c_ref, qr_ref, kr_ref):
        xv = x_ref[...]
        c_ref[...] = jnp.dot(xv, wdkv_ref[...], preferred_element_type=F32)
        qr_ref[...] = jnp.dot(xv, wqr_ref[...], preferred_element_type=F32)
        kr_ref[...] = jnp.dot(xv, wkr_ref[...], preferred_element_type=F32)

    return pl.pallas_call(
        body,
        out_shape=(
            jax.ShapeDtypeStruct((m, DC), F32),
            jax.ShapeDtypeStruct((m, H * Dr), F32),
            jax.ShapeDtypeStruct((m, Dr), F32),
        ),
    )(x2, Wdkv, Wqr, Wkr)


def _gather_qkv(x2, c, Wuk, Wuv, Wq):
    s, d = x2.shape
    dc = c.shape[1]
    n_hops = N_Z - 1
    qchunk = d // N_Z

    def body(x_ref, c_ref, wuk_ref, wuv_ref, wq_ref,
             q_ref, k_ref, v_ref,
             cbuf, wukbuf, wuvbuf, send_sems, recv_sems):
        mx = lax.axis_index("x")
        my = lax.axis_index("y")
        mz = lax.axis_index("z")
        left = (mz + N_Z - 1) % N_Z
        right = (mz + 1) % N_Z

        barrier = pltpu.get_barrier_semaphore()
        for nbr in (left, right):
            pl.semaphore_signal(
                barrier, inc=1,
                device_id=(mx, my, nbr),
                device_id_type=pl.DeviceIdType.MESH,
            )
        pl.semaphore_wait(barrier, 2)

        def start_hop(h, srcs):
            rdmas = []
            for t, (src, buf) in enumerate(zip(srcs, (cbuf, wukbuf, wuvbuf))):
                rdma = pltpu.make_async_remote_copy(
                    src_ref=src,
                    dst_ref=buf.at[h],
                    send_sem=send_sems.at[h, t],
                    recv_sem=recv_sems.at[h, t],
                    device_id=(mx, my, right),
                    device_id_type=pl.DeviceIdType.MESH,
                )
                rdma.start()
                rdmas.append(rdma)
            return rdmas

        rdmas = start_hop(0, (c_ref, wuk_ref, wuv_ref))
        k_ref[...] = jnp.dot(c_ref[...], wuk_ref[...], preferred_element_type=F32)
        v_ref[...] = jnp.dot(c_ref[...], wuv_ref[...], preferred_element_type=F32)
        q_ref[:, 0:qchunk] = jnp.dot(
            x_ref[...], wq_ref[:, 0:qchunk], preferred_element_type=F32)
        for r in rdmas:
            r.wait()

        for h in range(1, n_hops):
            rdmas = start_hop(h, (cbuf.at[h - 1], wukbuf.at[h - 1],
                                  wuvbuf.at[h - 1]))
            cv = cbuf[h - 1]
            k_ref[...] += jnp.dot(cv, wukbuf[h - 1], preferred_element_type=F32)
            v_ref[...] += jnp.dot(cv, wuvbuf[h - 1], preferred_element_type=F32)
            q_ref[:, h * qchunk:(h + 1) * qchunk] = jnp.dot(
                x_ref[...], wq_ref[:, h * qchunk:(h + 1) * qchunk],
                preferred_element_type=F32)
            for r in rdmas:
                r.wait()

        cv = cbuf[n_hops - 1]
        k_ref[...] += jnp.dot(cv, wukbuf[n_hops - 1], preferred_element_type=F32)
        v_ref[...] += jnp.dot(cv, wuvbuf[n_hops - 1], preferred_element_type=F32)
        q_ref[:, n_hops * qchunk:] = jnp.dot(
            x_ref[...], wq_ref[:, n_hops * qchunk:], preferred_element_type=F32)

    return pl.pallas_call(
        body,
        out_shape=(
            jax.ShapeDtypeStruct((s, d), F32),
            jax.ShapeDtypeStruct((s, d), F32),
            jax.ShapeDtypeStruct((s, d), F32),
        ),
        scratch_shapes=[
            pltpu.VMEM((n_hops, s, dc), F32),
            pltpu.VMEM((n_hops, dc, d), F32),
            pltpu.VMEM((n_hops, dc, d), F32),
            pltpu.SemaphoreType.DMA((n_hops, 3)),
            pltpu.SemaphoreType.DMA((n_hops, 3)),
        ],
        compiler_params=pltpu.CompilerParams(
            collective_id=0, vmem_limit_bytes=63 * 1024 * 1024),
    )(x2, c, Wuk, Wuv, Wq)


HB = 4


def _attention(q, k, v, qr, kr):
    s = q.shape[0]

    def body(q_ref, k_ref, v_ref, qr_ref, kr_ref, o_ref):
        kr_v = kr_ref[...]
        qr_v = qr_ref[...]
        for j in range(HB):
            q_h = q_ref[:, j * Dh:(j + 1) * Dh]
            k_h = k_ref[:, j * Dh:(j + 1) * Dh]
            qr_h = lax.slice(qr_v, (0, j * Dr), (s, (j + 1) * Dr))
            sc = lax.dot_general(
                q_h, k_h, (((1,), (1,)), ((), ())),
                preferred_element_type=F32,
            )
            sc += lax.dot_general(
                qr_h, kr_v, (((1,), (1,)), ((), ())),
                preferred_element_type=F32,
            )
            sc *= SCALE
            m = jnp.max(sc, axis=1, keepdims=True)
            p = jnp.exp(sc - m)
            p = p / jnp.sum(p, axis=1, keepdims=True)
            o_ref[:, j * Dh:(j + 1) * Dh] = jnp.dot(
                p, v_ref[:, j * Dh:(j + 1) * Dh], preferred_element_type=F32
            )

    return pl.pallas_call(
        body,
        grid=(H // HB,),
        in_specs=[
            pl.BlockSpec((s, HB * Dh), lambda g: (0, g)),
            pl.BlockSpec((s, HB * Dh), lambda g: (0, g)),
            pl.BlockSpec((s, HB * Dh), lambda g: (0, g)),
            pl.BlockSpec((s, HB * Dr), lambda g: (0, g)),
            pl.BlockSpec((s, Dr), lambda g: (0, 0)),
        ],
        out_specs=pl.BlockSpec((s, HB * Dh), lambda g: (0, g)),
        out_shape=jax.ShapeDtypeStruct((s, H * Dh), F32),
    )(q, k, v, qr, kr)


def kernel(x, Wdkv, Wuk, Wuv, Wq, Wqr, Wkr, Wo):
    b, s, d = x.shape
    x2 = x.reshape(s, d)

    c, qr, kr = _proj_lite(x2, Wdkv, Wqr, Wkr)
    q, k, v = _gather_qkv(x2, c, Wuk, Wuv, Wq)
    o = _attention(q, k, v, qr, kr)
    out = _matmul(o, Wo)
    return out.reshape(b, s, d)


# device time: 162772 ns/iter; 1.2595x vs baseline; 1.2595x over previous
import jax
import jax.numpy as jnp
from jax import lax
from jax.experimental import pallas as pl
from jax.experimental.pallas import tpu as pltpu

N_Z = 4
H, Dh, Dr = 16, 128, 32
DC = 128
SCALE = (Dh + Dr) ** -0.5
F32 = jnp.float32
BF16 = jnp.bfloat16


def _matmul(a, b):
    m, _ = a.shape
    _, n = b.shape

    def body(a_ref, b_ref, o_ref):
        o_ref[...] = jnp.dot(a_ref[...], b_ref[...], preferred_element_type=F32)

    return pl.pallas_call(
        body, out_shape=jax.ShapeDtypeStruct((m, n), F32),
        compiler_params=pltpu.CompilerParams(
            vmem_limit_bytes=63 * 1024 * 1024),
    )(a, b)


def _proj_lite(x2, Wdkv, Wqr, Wkr):
    m = x2.shape[0]

    def body(x_ref, wdkv_ref, wqr_ref, wkr_ref, c_ref, qr_ref, kr_ref):
        xv = x_ref[...]
        c_ref[...] = jnp.dot(xv, wdkv_ref[...], preferred_element_type=F32)
        qr_ref[...] = jnp.dot(xv, wqr_ref[...], preferred_element_type=F32)
        kr_ref[...] = jnp.dot(xv, wkr_ref[...], preferred_element_type=F32)

    return pl.pallas_call(
        body,
        out_shape=(
            jax.ShapeDtypeStruct((m, DC), F32),
            jax.ShapeDtypeStruct((m, H * Dr), F32),
            jax.ShapeDtypeStruct((m, Dr), F32),
        ),
    )(x2, Wdkv, Wqr, Wkr)


def _gather_qkv(x2, c, Wuk, Wuv, Wq):
    s, d = x2.shape
    dc = c.shape[1]
    n_hops = N_Z - 1
    qchunk = d // N_Z

    def body(x_ref, c_ref, wuk_ref, wuv_ref, wq_ref,
             q_ref, k_ref, v_ref,
             cbuf, wukbuf, wuvbuf, send_sems, recv_sems):
        mx = lax.axis_index("x")
        my = lax.axis_index("y")
        mz = lax.axis_index("z")
        left = (mz + N_Z - 1) % N_Z
        right = (mz + 1) % N_Z

        cbuf[n_hops] = c_ref[...].astype(BF16)
        wukbuf[n_hops] = wuk_ref[...].astype(BF16)
        wuvbuf[n_hops] = wuv_ref[...].astype(BF16)

        barrier = pltpu.get_barrier_semaphore()
        for nbr in (left, right):
            pl.semaphore_signal(
                barrier, inc=1,
                device_id=(mx, my, nbr),
                device_id_type=pl.DeviceIdType.MESH,
            )
        pl.semaphore_wait(barrier, 2)

        def start_hop(h, src_slot):
            rdmas = []
            for t, buf in enumerate((cbuf, wukbuf, wuvbuf)):
                rdma = pltpu.make_async_remote_copy(
                    src_ref=buf.at[src_slot],
                    dst_ref=buf.at[h],
                    send_sem=send_sems.at[h, t],
                    recv_sem=recv_sems.at[h, t],
                    device_id=(mx, my, right),
                    device_id_type=pl.DeviceIdType.MESH,
                )
                rdma.start()
                rdmas.append(rdma)
            return rdmas

        rdmas = start_hop(0, n_hops)
        k_ref[...] = jnp.dot(c_ref[...], wuk_ref[...], preferred_element_type=F32)
        v_ref[...] = jnp.dot(c_ref[...], wuv_ref[...], preferred_element_type=F32)
        q_ref[:, 0:qchunk] = jnp.dot(
            x_ref[...], wq_ref[:, 0:qchunk], preferred_element_type=F32)
        for r in rdmas:
            r.wait()

        for h in range(1, n_hops):
            rdmas = start_hop(h, h - 1)
            cv = cbuf[h - 1]
            k_ref[...] += jnp.dot(cv, wukbuf[h - 1], preferred_element_type=F32)
            v_ref[...] += jnp.dot(cv, wuvbuf[h - 1], preferred_element_type=F32)
            q_ref[:, h * qchunk:(h + 1) * qchunk] = jnp.dot(
                x_ref[...], wq_ref[:, h * qchunk:(h + 1) * qchunk],
                preferred_element_type=F32)
            for r in rdmas:
                r.wait()

        cv = cbuf[n_hops - 1]
        k_ref[...] += jnp.dot(cv, wukbuf[n_hops - 1], preferred_element_type=F32)
        v_ref[...] += jnp.dot(cv, wuvbuf[n_hops - 1], preferred_element_type=F32)
        q_ref[:, n_hops * qchunk:] = jnp.dot(
            x_ref[...], wq_ref[:, n_hops * qchunk:], preferred_element_type=F32)

    return pl.pallas_call(
        body,
        out_shape=(
            jax.ShapeDtypeStruct((s, d), F32),
            jax.ShapeDtypeStruct((s, d), F32),
            jax.ShapeDtypeStruct((s, d), F32),
        ),
        scratch_shapes=[
            pltpu.VMEM((N_Z, s, dc), BF16),
            pltpu.VMEM((N_Z, dc, d), BF16),
            pltpu.VMEM((N_Z, dc, d), BF16),
            pltpu.SemaphoreType.DMA((n_hops, 3)),
            pltpu.SemaphoreType.DMA((n_hops, 3)),
        ],
        compiler_params=pltpu.CompilerParams(
            collective_id=0, vmem_limit_bytes=63 * 1024 * 1024),
    )(x2, c, Wuk, Wuv, Wq)


HB = 4


def _attention(q, k, v, qr, kr):
    s = q.shape[0]

    def body(q_ref, k_ref, v_ref, qr_ref, kr_ref, o_ref):
        kr_v = kr_ref[...]
        qr_v = qr_ref[...]
        for j in range(HB):
            q_h = q_ref[:, j * Dh:(j + 1) * Dh]
            k_h = k_ref[:, j * Dh:(j + 1) * Dh]
            qr_h = lax.slice(qr_v, (0, j * Dr), (s, (j + 1) * Dr))
            sc = lax.dot_general(
                q_h, k_h, (((1,), (1,)), ((), ())),
                preferred_element_type=F32,
            )
            sc += lax.dot_general(
                qr_h, kr_v, (((1,), (1,)), ((), ())),
                preferred_element_type=F32,
            )
            sc *= SCALE
            m = jnp.max(sc, axis=1, keepdims=True)
            p = jnp.exp(sc - m)
            p = p / jnp.sum(p, axis=1, keepdims=True)
            o_ref[:, j * Dh:(j + 1) * Dh] = jnp.dot(
                p, v_ref[:, j * Dh:(j + 1) * Dh], preferred_element_type=F32
            )

    return pl.pallas_call(
        body,
        grid=(H // HB,),
        in_specs=[
            pl.BlockSpec((s, HB * Dh), lambda g: (0, g)),
            pl.BlockSpec((s, HB * Dh), lambda g: (0, g)),
            pl.BlockSpec((s, HB * Dh), lambda g: (0, g)),
            pl.BlockSpec((s, HB * Dr), lambda g: (0, g)),
            pl.BlockSpec((s, Dr), lambda g: (0, 0)),
        ],
        out_specs=pl.BlockSpec((s, HB * Dh), lambda g: (0, g)),
        out_shape=jax.ShapeDtypeStruct((s, H * Dh), F32),
    )(q, k, v, qr, kr)


def kernel(x, Wdkv, Wuk, Wuv, Wq, Wqr, Wkr, Wo):
    b, s, d = x.shape
    x2 = x.reshape(s, d)

    c, qr, kr = _proj_lite(x2, Wdkv, Wqr, Wkr)
    q, k, v = _gather_qkv(x2, c, Wuk, Wuv, Wq)
    o = _attention(q, k, v, qr, kr)
    out = _matmul(o, Wo)
    return out.reshape(b, s, d)


# device time: 116480 ns/iter; 1.7600x vs baseline; 1.3974x over previous
import jax
import jax.numpy as jnp
from jax import lax
from jax.experimental import pallas as pl
from jax.experimental.pallas import tpu as pltpu

N_Z = 4
N_Q = 4
H, Dh, Dr = 16, 128, 32
HQ = H // N_Q
DC = 128
SCALE = (Dh + Dr) ** -0.5
F32 = jnp.float32
BF16 = jnp.bfloat16


def _proj_lite(x2, Wdkv, Wqr, Wkr):
    m = x2.shape[0]

    def body(x_ref, wdkv_ref, wqr_ref, wkr_ref, c_ref, qr_ref, kr_ref):
        xv = x_ref[...]
        c_ref[...] = jnp.dot(xv, wdkv_ref[...], preferred_element_type=F32)
        qr_ref[...] = jnp.dot(xv, wqr_ref[...], preferred_element_type=F32)
        kr_ref[...] = jnp.dot(xv, wkr_ref[...], preferred_element_type=F32)

    return pl.pallas_call(
        body,
        out_shape=(
            jax.ShapeDtypeStruct((m, DC), F32),
            jax.ShapeDtypeStruct((m, H * Dr), F32),
            jax.ShapeDtypeStruct((m, Dr), F32),
        ),
    )(x2, Wdkv, Wqr, Wkr)


def _gather_qkv(x2, c, Wuk, Wuv, Wq):
    s, d = x2.shape
    dq = d // N_Q
    n_hops = N_Z - 1
    qchunk = dq // N_Z

    def body(x_ref, c_ref, wuk_ref, wuv_ref, wq_ref,
             q_ref, k_ref, v_ref,
             cbuf, wukbuf, wuvbuf, wqstage, kacc, vacc,
             send_sems, recv_sems):
        mx = lax.axis_index("x")
        my = lax.axis_index("y")
        mz = lax.axis_index("z")
        hq = mx * 2 + my
        left = (mz + N_Z - 1) % N_Z
        right = (mz + 1) % N_Z

        for qq in range(N_Q):
            @pl.when(hq == qq)
            def _():
                wukbuf[n_hops] = wuk_ref[:, qq * dq:(qq + 1) * dq].astype(BF16)
                wuvbuf[n_hops] = wuv_ref[:, qq * dq:(qq + 1) * dq].astype(BF16)
                wqstage[...] = wq_ref[:, qq * dq:(qq + 1) * dq]
        cbuf[n_hops] = c_ref[...].astype(BF16)

        barrier = pltpu.get_barrier_semaphore()
        for nbr in (left, right):
            pl.semaphore_signal(
                barrier, inc=1,
                device_id=(mx, my, nbr),
                device_id_type=pl.DeviceIdType.MESH,
            )
        pl.semaphore_wait(barrier, 2)

        def start_hop(h, src_slot):
            rdmas = []
            for t, buf in enumerate((cbuf, wukbuf, wuvbuf)):
                rdma = pltpu.make_async_remote_copy(
                    src_ref=buf.at[src_slot],
                    dst_ref=buf.at[h],
                    send_sem=send_sems.at[h, t],
                    recv_sem=recv_sems.at[h, t],
                    device_id=(mx, my, right),
                    device_id_type=pl.DeviceIdType.MESH,
                )
                rdma.start()
                rdmas.append(rdma)
            return rdmas

        def fold(slot):
            cv = cbuf[slot]
            kacc[...] += jnp.dot(cv, wukbuf[slot], preferred_element_type=F32)
            vacc[...] += jnp.dot(cv, wuvbuf[slot], preferred_element_type=F32)

        rdmas = start_hop(0, n_hops)
        cv = cbuf[n_hops]
        kacc[...] = jnp.dot(cv, wukbuf[n_hops], preferred_element_type=F32)
        vacc[...] = jnp.dot(cv, wuvbuf[n_hops], preferred_element_type=F32)
        q_ref[:, 0:qchunk] = jnp.dot(
            x_ref[...], wqstage[:, 0:qchunk],
            preferred_element_type=F32).astype(BF16)
        for r in rdmas:
            r.wait()

        for h in range(1, n_hops):
            rdmas = start_hop(h, h - 1)
            fold(h - 1)
            q_ref[:, h * qchunk:(h + 1) * qchunk] = jnp.dot(
                x_ref[...], wqstage[:, h * qchunk:(h + 1) * qchunk],
                preferred_element_type=F32).astype(BF16)
            for r in rdmas:
                r.wait()

        fold(n_hops - 1)
        q_ref[:, n_hops * qchunk:] = jnp.dot(
            x_ref[...], wqstage[:, n_hops * qchunk:],
            preferred_element_type=F32).astype(BF16)
        k_ref[...] = kacc[...].astype(BF16)
        v_ref[...] = vacc[...].astype(BF16)

    return pl.pallas_call(
        body,
        out_shape=(
            jax.ShapeDtypeStruct((s, dq), BF16),
            jax.ShapeDtypeStruct((s, dq), BF16),
            jax.ShapeDtypeStruct((s, dq), BF16),
        ),
        scratch_shapes=[
            pltpu.VMEM((N_Z, s, DC), BF16),
            pltpu.VMEM((N_Z, DC, dq), BF16),
            pltpu.VMEM((N_Z, DC, dq), BF16),
            pltpu.VMEM((d, dq), F32),
            pltpu.VMEM((s, dq), F32),
            pltpu.VMEM((s, dq), F32),
            pltpu.SemaphoreType.DMA((n_hops, 3)),
            pltpu.SemaphoreType.DMA((n_hops, 3)),
        ],
        compiler_params=pltpu.CompilerParams(
            collective_id=0, vmem_limit_bytes=63 * 1024 * 1024),
    )(x2, c, Wuk, Wuv, Wq)


def _attn_bcast_out(q, k, v, qr, kr, Wo):
    s, dq = q.shape
    d = Wo.shape[0]

    def body(q_ref, k_ref, v_ref, qr_ref, kr_ref, wo_ref, out_ref,
             oq, oall, qrq, send_sems, recv_sems):
        mx = lax.axis_index("x")
        my = lax.axis_index("y")
        mz = lax.axis_index("z")
        hq = mx * 2 + my

        for qq in range(N_Q):
            @pl.when(hq == qq)
            def _():
                qrq[...] = qr_ref[:, qq * HQ * Dr:(qq + 1) * HQ * Dr]

        kr_v = kr_ref[...]
        for j in range(HQ):
            q_h = q_ref[:, j * Dh:(j + 1) * Dh]
            k_h = k_ref[:, j * Dh:(j + 1) * Dh]
            qr_h = qrq[:, j * Dr:(j + 1) * Dr]
            sc = lax.dot_general(
                q_h, k_h, (((1,), (1,)), ((), ())),
                preferred_element_type=F32,
            )
            sc += lax.dot_general(
                qr_h, kr_v, (((1,), (1,)), ((), ())),
                preferred_element_type=F32,
            )
            sc *= SCALE
            m = jnp.max(sc, axis=1, keepdims=True)
            p = jnp.exp(sc - m)
            p = p / jnp.sum(p, axis=1, keepdims=True)
            oq[:, j * Dh:(j + 1) * Dh] = jnp.dot(
                p, v_ref[:, j * Dh:(j + 1) * Dh],
                preferred_element_type=F32).astype(BF16)

        barrier = pltpu.get_barrier_semaphore()
        peers = []
        for i in (1, 2, 3):
            p_hq = hq ^ i
            px = p_hq // 2
            py = p_hq % 2
            peers.append((p_hq, px, py))
            pl.semaphore_signal(
                barrier, inc=1,
                device_id=(px, py, mz),
                device_id_type=pl.DeviceIdType.MESH,
            )
        pl.semaphore_wait(barrier, 3)

        sends = []
        for i, (p_hq, px, py) in enumerate(peers):
            rdma = pltpu.make_async_remote_copy(
                src_ref=oq,
                dst_ref=oall.at[hq],
                send_sem=send_sems.at[i],
                recv_sem=recv_sems.at[i],
                device_id=(px, py, mz),
                device_id_type=pl.DeviceIdType.MESH,
            )
            rdma.start()
            sends.append(rdma)

        out_ref[...] = lax.dot_general(
            oq[...], wo_ref[pl.ds(hq * dq, dq), :],
            (((1,), (0,)), ((), ())), preferred_element_type=F32)

        for i, (p_hq, px, py) in enumerate(peers):
            recv = pltpu.make_async_remote_copy(
                src_ref=oq,
                dst_ref=oall.at[p_hq],
                send_sem=send_sems.at[i],
                recv_sem=recv_sems.at[i],
                device_id=(px, py, mz),
                device_id_type=pl.DeviceIdType.MESH,
            )
            recv.wait_recv()
            out_ref[...] += lax.dot_general(
                oall[p_hq], wo_ref[pl.ds(p_hq * dq, dq), :],
                (((1,), (0,)), ((), ())), preferred_element_type=F32)
        for rdma in sends:
            rdma.wait_send()

    return pl.pallas_call(
        body,
        out_shape=jax.ShapeDtypeStruct((s, d), F32),
        scratch_shapes=[
            pltpu.VMEM((s, dq), BF16),
            pltpu.VMEM((N_Q, s, dq), BF16),
            pltpu.VMEM((s, HQ * Dr), F32),
            pltpu.SemaphoreType.DMA((3,)),
            pltpu.SemaphoreType.DMA((3,)),
        ],
        compiler_params=pltpu.CompilerParams(
            collective_id=1, vmem_limit_bytes=63 * 1024 * 1024),
    )(q, k, v, qr, kr, Wo)


def kernel(x, Wdkv, Wuk, Wuv, Wq, Wqr, Wkr, Wo):
    b, s, d = x.shape
    x2 = x.reshape(s, d)

    c, qr, kr = _proj_lite(x2, Wdkv, Wqr, Wkr)
    q, k, v = _gather_qkv(x2, c, Wuk, Wuv, Wq)
    out = _attn_bcast_out(q, k, v, qr, kr, Wo)
    return out.reshape(b, s, d)


# device time: 101822 ns/iter; 2.0134x vs baseline; 1.1440x over previous
import jax
import jax.numpy as jnp
from jax import lax
from jax.experimental import pallas as pl
from jax.experimental.pallas import tpu as pltpu

N_Z = 4
N_Q = 4
H, Dh, Dr = 16, 128, 32
HQ = H // N_Q
DC = 128
SCALE = (Dh + Dr) ** -0.5
F32 = jnp.float32
BF16 = jnp.bfloat16


def _gather_qkv(x2, Wdkv, Wuk, Wuv, Wq, Wqr, Wkr):
    s, d = x2.shape
    dq = d // N_Q
    n_hops = N_Z - 1
    qchunk = dq // N_Z

    def body(x_ref, wdkv_ref, wuk_ref, wuv_ref, wq_ref, wqr_ref, wkr_ref,
             q_ref, k_ref, v_ref, qrq_ref, kr_ref,
             cbuf, wukbuf, wuvbuf, wqstage, wqrstage, kacc, vacc,
             send_sems, recv_sems, wq_sem):
        mx = lax.axis_index("x")
        my = lax.axis_index("y")
        mz = lax.axis_index("z")
        hq = mx * 2 + my
        left = (mz + N_Z - 1) % N_Z
        right = (mz + 1) % N_Z

        wq_copy = pltpu.make_async_copy(
            wq_ref.at[:, pl.ds(hq * dq, dq)], wqstage, wq_sem)
        wq_copy.start()

        for qq in range(N_Q):
            @pl.when(hq == qq)
            def _():
                wukbuf[n_hops] = wuk_ref[:, qq * dq:(qq + 1) * dq].astype(BF16)
                wuvbuf[n_hops] = wuv_ref[:, qq * dq:(qq + 1) * dq].astype(BF16)
                wqrstage[...] = wqr_ref[:, qq * HQ * Dr:(qq + 1) * HQ * Dr]
        xv = x_ref[...]
        cbuf[n_hops] = jnp.dot(
            xv, wdkv_ref[...], preferred_element_type=F32).astype(BF16)

        barrier = pltpu.get_barrier_semaphore()
        for nbr in (left, right):
            pl.semaphore_signal(
                barrier, inc=1,
                device_id=(mx, my, nbr),
                device_id_type=pl.DeviceIdType.MESH,
            )
        pl.semaphore_wait(barrier, 2)

        def start_hop(h, src_slot):
            rdmas = []
            for t, buf in enumerate((cbuf, wukbuf, wuvbuf)):
                rdma = pltpu.make_async_remote_copy(
                    src_ref=buf.at[src_slot],
                    dst_ref=buf.at[h],
                    send_sem=send_sems.at[h, t],
                    recv_sem=recv_sems.at[h, t],
                    device_id=(mx, my, right),
                    device_id_type=pl.DeviceIdType.MESH,
                )
                rdma.start()
                rdmas.append(rdma)
            return rdmas

        def fold(slot, first=False):
            cv = cbuf[slot]
            kp = jnp.dot(cv, wukbuf[slot], preferred_element_type=F32)
            vp = jnp.dot(cv, wuvbuf[slot], preferred_element_type=F32)
            if first:
                kacc[...] = kp
                vacc[...] = vp
            else:
                kacc[...] += kp
                vacc[...] += vp

        rdmas = start_hop(0, n_hops)
        fold(n_hops, first=True)
        qrq_ref[...] = jnp.dot(xv, wqrstage[...], preferred_element_type=F32)
        kr_ref[...] = jnp.dot(xv, wkr_ref[...], preferred_element_type=F32)
        wq_copy.wait()
        q_ref[:, 0:qchunk] = jnp.dot(
            xv, wqstage[:, 0:qchunk], preferred_element_type=F32).astype(BF16)
        for r in rdmas:
            r.wait()

        for h in range(1, n_hops):
            rdmas = start_hop(h, h - 1)
            fold(h - 1)
            q_ref[:, h * qchunk:(h + 1) * qchunk] = jnp.dot(
                xv, wqstage[:, h * qchunk:(h + 1) * qchunk],
                preferred_element_type=F32).astype(BF16)
            for r in rdmas:
                r.wait()

        fold(n_hops - 1)
        q_ref[:, n_hops * qchunk:] = jnp.dot(
            xv, wqstage[:, n_hops * qchunk:],
            preferred_element_type=F32).astype(BF16)
        k_ref[...] = kacc[...].astype(BF16)
        v_ref[...] = vacc[...].astype(BF16)

    vm = pl.BlockSpec(memory_space=pltpu.VMEM)
    return pl.pallas_call(
        body,
        in_specs=[vm, vm, vm, vm,
                  pl.BlockSpec(memory_space=pl.ANY), vm, vm],
        out_shape=(
            jax.ShapeDtypeStruct((s, dq), BF16),
            jax.ShapeDtypeStruct((s, dq), BF16),
            jax.ShapeDtypeStruct((s, dq), BF16),
            jax.ShapeDtypeStruct((s, HQ * Dr), F32),
            jax.ShapeDtypeStruct((s, Dr), F32),
        ),
        scratch_shapes=[
            pltpu.VMEM((N_Z, s, DC), BF16),
            pltpu.VMEM((N_Z, DC, dq), BF16),
            pltpu.VMEM((N_Z, DC, dq), BF16),
            pltpu.VMEM((d, dq), F32),
            pltpu.VMEM((d, HQ * Dr), F32),
            pltpu.VMEM((s, dq), F32),
            pltpu.VMEM((s, dq), F32),
            pltpu.SemaphoreType.DMA((n_hops, 3)),
            pltpu.SemaphoreType.DMA((n_hops, 3)),
            pltpu.SemaphoreType.DMA,
        ],
        compiler_params=pltpu.CompilerParams(
            collective_id=0, vmem_limit_bytes=63 * 1024 * 1024),
    )(x2, Wdkv, Wuk, Wuv, Wq, Wqr, Wkr)


def _attn_bcast_out(q, k, v, qrq, kr, Wo):
    s, dq = q.shape
    d = Wo.shape[0]

    def body(q_ref, k_ref, v_ref, qrq_ref, kr_ref, wo_ref, out_ref,
             oq, oall, send_sems, recv_sems):
        mx = lax.axis_index("x")
        my = lax.axis_index("y")
        mz = lax.axis_index("z")
        hq = mx * 2 + my

        barrier = pltpu.get_barrier_semaphore()
        peers = []
        for i in (1, 2, 3):
            p_hq = hq ^ i
            px = p_hq // 2
            py = p_hq % 2
            peers.append((p_hq, px, py))
            pl.semaphore_signal(
                barrier, inc=1,
                device_id=(px, py, mz),
                device_id_type=pl.DeviceIdType.MESH,
            )
        pl.semaphore_wait(barrier, 3)

        kr_v = kr_ref[...]
        sends = []
        for j in range(HQ):
            q_h = q_ref[:, j * Dh:(j + 1) * Dh]
            k_h = k_ref[:, j * Dh:(j + 1) * Dh]
            qr_h = qrq_ref[:, j * Dr:(j + 1) * Dr]
            sc = lax.dot_general(
                q_h, k_h, (((1,), (1,)), ((), ())),
                preferred_element_type=F32,
            )
            sc += lax.dot_general(
                qr_h, kr_v, (((1,), (1,)), ((), ())),
                preferred_element_type=F32,
            )
            sc *= SCALE
            m = jnp.max(sc, axis=1, keepdims=True)
            p = jnp.exp(sc - m)
            p = p / jnp.sum(p, axis=1, keepdims=True)
            oq[:, j * Dh:(j + 1) * Dh] = jnp.dot(
                p, v_ref[:, j * Dh:(j + 1) * Dh],
                preferred_element_type=F32).astype(BF16)
            for i, (p_hq, px, py) in enumerate(peers):
                rdma = pltpu.make_async_remote_copy(
                    src_ref=oq.at[:, pl.ds(j * Dh, Dh)],
                    dst_ref=oall.at[hq, :, pl.ds(j * Dh, Dh)],
                    send_sem=send_sems.at[i, j],
                    recv_sem=recv_sems.at[i, j],
                    device_id=(px, py, mz),
                    device_id_type=pl.DeviceIdType.MESH,
                )
                rdma.start()
                sends.append(rdma)

        out_ref[...] = lax.dot_general(
            oq[...], wo_ref[pl.ds(hq * dq, dq), :],
            (((1,), (0,)), ((), ())), preferred_element_type=F32)

        for i, (p_hq, px, py) in enumerate(peers):
            for j in range(HQ):
                recv = pltpu.make_async_remote_copy(
                    src_ref=oq.at[:, pl.ds(j * Dh, Dh)],
                    dst_ref=oall.at[p_hq, :, pl.ds(j * Dh, Dh)],
                    send_sem=send_sems.at[i, j],
                    recv_sem=recv_sems.at[i, j],
                    device_id=(px, py, mz),
                    device_id_type=pl.DeviceIdType.MESH,
                )
                recv.wait_recv()
                out_ref[...] += lax.dot_general(
                    oall[p_hq, :, j * Dh:(j + 1) * Dh],
                    wo_ref[pl.ds(p_hq * dq + j * Dh, Dh), :],
                    (((1,), (0,)), ((), ())), preferred_element_type=F32)
        for rdma in sends:
            rdma.wait_send()

    return pl.pallas_call(
        body,
        out_shape=jax.ShapeDtypeStruct((s, d), F32),
        scratch_shapes=[
            pltpu.VMEM((s, dq), BF16),
            pltpu.VMEM((N_Q, s, dq), BF16),
            pltpu.SemaphoreType.DMA((3, HQ)),
            pltpu.SemaphoreType.DMA((3, HQ)),
        ],
        compiler_params=pltpu.CompilerParams(
            collective_id=1, vmem_limit_bytes=63 * 1024 * 1024),
    )(q, k, v, qrq, kr, Wo)


def kernel(x, Wdkv, Wuk, Wuv, Wq, Wqr, Wkr, Wo):
    b, s, d = x.shape
    x2 = x.reshape(s, d)

    q, k, v, qrq, kr = _gather_qkv(x2, Wdkv, Wuk, Wuv, Wq, Wqr, Wkr)
    out = _attn_bcast_out(q, k, v, qrq, kr, Wo)
    return out.reshape(b, s, d)


# device time: 91612 ns/iter; 2.2378x vs baseline; 1.1114x over previous
import jax
import jax.numpy as jnp
from jax import lax
from jax.experimental import pallas as pl
from jax.experimental.pallas import tpu as pltpu

N_Z = 4
N_Q = 4
H, Dh, Dr = 16, 128, 32
HQ = H // N_Q
DC = 128
SCALE = (Dh + Dr) ** -0.5
F32 = jnp.float32
BF16 = jnp.bfloat16


def _gather_qkv(x2, Wdkv, Wuk, Wuv, Wq, Wqr, Wkr):
    s, d = x2.shape
    dq = d // N_Q
    n_hops = N_Z - 1
    qchunk = dq // N_Z

    def body(x_ref, wdkv_ref, wuk_ref, wuv_ref, wq_ref, wqr_ref, wkr_ref,
             q_ref, k_ref, v_ref, qrq_ref, kr_ref,
             cbuf, wukbuf, wuvbuf, wqstage, wqrstage, kacc, vacc,
             send_sems, recv_sems, wq_sem):
        mx = lax.axis_index("x")
        my = lax.axis_index("y")
        mz = lax.axis_index("z")
        hq = mx * 2 + my
        left = (mz + N_Z - 1) % N_Z
        right = (mz + 1) % N_Z

        wq_copy = pltpu.make_async_copy(
            wq_ref.at[:, pl.ds(hq * dq, dq)], wqstage, wq_sem)
        wq_copy.start()

        for qq in range(N_Q):
            @pl.when(hq == qq)
            def _():
                wukbuf[n_hops] = wuk_ref[:, qq * dq:(qq + 1) * dq].astype(BF16)
                wuvbuf[n_hops] = wuv_ref[:, qq * dq:(qq + 1) * dq].astype(BF16)
                wqrstage[...] = wqr_ref[:, qq * HQ * Dr:(qq + 1) * HQ * Dr]
        xv = x_ref[...]
        cbuf[n_hops] = jnp.dot(
            xv, wdkv_ref[...], preferred_element_type=F32).astype(BF16)

        barrier = pltpu.get_barrier_semaphore()
        for nbr in (left, right):
            pl.semaphore_signal(
                barrier, inc=1,
                device_id=(mx, my, nbr),
                device_id_type=pl.DeviceIdType.MESH,
            )
        pl.semaphore_wait(barrier, 2)

        def start_hop(h, src_slot):
            rdmas = []
            for t, buf in enumerate((cbuf, wukbuf, wuvbuf)):
                rdma = pltpu.make_async_remote_copy(
                    src_ref=buf.at[src_slot],
                    dst_ref=buf.at[h],
                    send_sem=send_sems.at[h, t],
                    recv_sem=recv_sems.at[h, t],
                    device_id=(mx, my, right),
                    device_id_type=pl.DeviceIdType.MESH,
                )
                rdma.start()
                rdmas.append(rdma)
            return rdmas

        def fold(slot, first=False):
            cv = cbuf[slot]
            kp = jnp.dot(cv, wukbuf[slot], preferred_element_type=F32)
            vp = jnp.dot(cv, wuvbuf[slot], preferred_element_type=F32)
            if first:
                kacc[...] = kp
                vacc[...] = vp
            else:
                kacc[...] += kp
                vacc[...] += vp

        rdmas = start_hop(0, n_hops)
        fold(n_hops, first=True)
        qrq_ref[...] = jnp.dot(xv, wqrstage[...], preferred_element_type=F32)
        kr_ref[...] = jnp.dot(xv, wkr_ref[...], preferred_element_type=F32)
        wq_copy.wait()
        q_ref[:, 0:qchunk] = jnp.dot(
            xv, wqstage[:, 0:qchunk], preferred_element_type=F32).astype(BF16)
        for r in rdmas:
            r.wait()

        for h in range(1, n_hops):
            rdmas = start_hop(h, h - 1)
            fold(h - 1)
            q_ref[:, h * qchunk:(h + 1) * qchunk] = jnp.dot(
                xv, wqstage[:, h * qchunk:(h + 1) * qchunk],
                preferred_element_type=F32).astype(BF16)
            for r in rdmas:
                r.wait()

        fold(n_hops - 1)
        q_ref[:, n_hops * qchunk:] = jnp.dot(
            xv, wqstage[:, n_hops * qchunk:],
            preferred_element_type=F32).astype(BF16)
        k_ref[...] = kacc[...].astype(BF16)
        v_ref[...] = vacc[...].astype(BF16)

    vm = pl.BlockSpec(memory_space=pltpu.VMEM)
    return pl.pallas_call(
        body,
        in_specs=[vm, vm, vm, vm,
                  pl.BlockSpec(memory_space=pl.ANY), vm, vm],
        out_shape=(
            jax.ShapeDtypeStruct((s, dq), BF16),
            jax.ShapeDtypeStruct((s, dq), BF16),
            jax.ShapeDtypeStruct((s, dq), BF16),
            jax.ShapeDtypeStruct((s, HQ * Dr), F32),
            jax.ShapeDtypeStruct((s, Dr), F32),
        ),
        scratch_shapes=[
            pltpu.VMEM((N_Z, s, DC), BF16),
            pltpu.VMEM((N_Z, DC, dq), BF16),
            pltpu.VMEM((N_Z, DC, dq), BF16),
            pltpu.VMEM((d, dq), F32),
            pltpu.VMEM((d, HQ * Dr), F32),
            pltpu.VMEM((s, dq), F32),
            pltpu.VMEM((s, dq), F32),
            pltpu.SemaphoreType.DMA((n_hops, 3)),
            pltpu.SemaphoreType.DMA((n_hops, 3)),
            pltpu.SemaphoreType.DMA,
        ],
        compiler_params=pltpu.CompilerParams(
            collective_id=0, vmem_limit_bytes=63 * 1024 * 1024),
    )(x2, Wdkv, Wuk, Wuv, Wq, Wqr, Wkr)


def _attn_bcast_out(q, k, v, qrq, kr, Wo):
    s, dq = q.shape
    d = Wo.shape[0]

    def body(q_ref, k_ref, v_ref, qrq_ref, kr_ref, wo_ref, out_ref,
             oq, oall, wostage, send_sems, recv_sems, wo_sem):
        mx = lax.axis_index("x")
        my = lax.axis_index("y")
        mz = lax.axis_index("z")
        hq = mx * 2 + my

        wo_copy = pltpu.make_async_copy(wo_ref, wostage, wo_sem)
        wo_copy.start()

        barrier = pltpu.get_barrier_semaphore()
        peers = []
        for i in (1, 2, 3):
            p_hq = hq ^ i
            px = p_hq // 2
            py = p_hq % 2
            peers.append((p_hq, px, py))
            pl.semaphore_signal(
                barrier, inc=1,
                device_id=(px, py, mz),
                device_id_type=pl.DeviceIdType.MESH,
            )
        pl.semaphore_wait(barrier, 3)

        kr_v = kr_ref[...]
        sends = []
        for j in range(HQ):
            q_h = q_ref[:, j * Dh:(j + 1) * Dh]
            k_h = k_ref[:, j * Dh:(j + 1) * Dh]
            qr_h = qrq_ref[:, j * Dr:(j + 1) * Dr]
            sc = lax.dot_general(
                q_h, k_h, (((1,), (1,)), ((), ())),
                preferred_element_type=F32,
            )
            sc += lax.dot_general(
                qr_h, kr_v, (((1,), (1,)), ((), ())),
                preferred_element_type=F32,
            )
            sc *= SCALE
            m = jnp.max(sc, axis=1, keepdims=True)
            p = jnp.exp(sc - m)
            p = p / jnp.sum(p, axis=1, keepdims=True)
            oq[:, j * Dh:(j + 1) * Dh] = jnp.dot(
                p, v_ref[:, j * Dh:(j + 1) * Dh],
                preferred_element_type=F32).astype(BF16)
            for i, (p_hq, px, py) in enumerate(peers):
                rdma = pltpu.make_async_remote_copy(
                    src_ref=oq.at[:, pl.ds(j * Dh, Dh)],
                    dst_ref=oall.at[hq, :, pl.ds(j * Dh, Dh)],
                    send_sem=send_sems.at[i, j],
                    recv_sem=recv_sems.at[i, j],
                    device_id=(px, py, mz),
                    device_id_type=pl.DeviceIdType.MESH,
                )
                rdma.start()
                sends.append(rdma)

        wo_copy.wait()
        out_ref[...] = lax.dot_general(
            oq[...], wostage[pl.ds(hq * dq, dq), :],
            (((1,), (0,)), ((), ())), preferred_element_type=F32)

        for i, (p_hq, px, py) in enumerate(peers):
            for j in range(HQ):
                recv = pltpu.make_async_remote_copy(
                    src_ref=oq.at[:, pl.ds(j * Dh, Dh)],
                    dst_ref=oall.at[p_hq, :, pl.ds(j * Dh, Dh)],
                    send_sem=send_sems.at[i, j],
                    recv_sem=recv_sems.at[i, j],
                    device_id=(px, py, mz),
                    device_id_type=pl.DeviceIdType.MESH,
                )
                recv.wait_recv()
            out_ref[...] += lax.dot_general(
                oall[p_hq],
                wostage[pl.ds(p_hq * dq, dq), :],
                (((1,), (0,)), ((), ())), preferred_element_type=F32)
        for rdma in sends:
            rdma.wait_send()

    vm = pl.BlockSpec(memory_space=pltpu.VMEM)
    return pl.pallas_call(
        body,
        in_specs=[vm, vm, vm, vm, vm, pl.BlockSpec(memory_space=pl.ANY)],
        out_shape=jax.ShapeDtypeStruct((s, d), F32),
        scratch_shapes=[
            pltpu.VMEM((s, dq), BF16),
            pltpu.VMEM((N_Q, s, dq), BF16),
            pltpu.VMEM((d, d), F32),
            pltpu.SemaphoreType.DMA((3, HQ)),
            pltpu.SemaphoreType.DMA((3, HQ)),
            pltpu.SemaphoreType.DMA,
        ],
        compiler_params=pltpu.CompilerParams(
            collective_id=1, vmem_limit_bytes=63 * 1024 * 1024),
    )(q, k, v, qrq, kr, Wo)


def kernel(x, Wdkv, Wuk, Wuv, Wq, Wqr, Wkr, Wo):
    b, s, d = x.shape
    x2 = x.reshape(s, d)

    q, k, v, qrq, kr = _gather_qkv(x2, Wdkv, Wuk, Wuv, Wq, Wqr, Wkr)
    out = _attn_bcast_out(q, k, v, qrq, kr, Wo)
    return out.reshape(b, s, d)


# device time: 88683 ns/iter; 2.3117x vs baseline; 1.0330x over previous
import jax
import jax.numpy as jnp
from jax import lax
from jax.experimental import pallas as pl
from jax.experimental.pallas import tpu as pltpu

N_Z = 4
N_Q = 4
H, Dh, Dr = 16, 128, 32
HQ = H // N_Q
DC = 128
SCALE = (Dh + Dr) ** -0.5
F32 = jnp.float32
BF16 = jnp.bfloat16


def _mla_fused(x2, Wdkv, Wuk, Wuv, Wq, Wqr, Wkr, Wo):
    s, d = x2.shape
    dq = d // N_Q
    n_hops = N_Z - 1
    qchunk = dq // N_Z

    def body(x_ref, wdkv_ref, wuk_ref, wuv_ref, wq_ref, wqr_ref, wkr_ref,
             wo_ref, out_ref,
             cbuf, wukbuf, wuvbuf, wqstage, wqrstage,
             qs, kacc, vacc, qrq, krs, oq, oall, wo2,
             send_sems, recv_sems, wq_sem,
             osend_sems, orecv_sems, wo_sems, xybar):
        mx = lax.axis_index("x")
        my = lax.axis_index("y")
        mz = lax.axis_index("z")
        hq = mx * 2 + my
        left = (mz + N_Z - 1) % N_Z
        right = (mz + 1) % N_Z

        wq_copy = pltpu.make_async_copy(
            wq_ref.at[:, pl.ds(hq * dq, dq)], wqstage, wq_sem)
        wq_copy.start()
        wo_own = pltpu.make_async_copy(
            wo_ref.at[pl.ds(hq * dq, dq), :], wo2.at[0], wo_sems.at[0])
        wo_own.start()

        for qq in range(N_Q):
            @pl.when(hq == qq)
            def _():
                wukbuf[n_hops] = wuk_ref[:, qq * dq:(qq + 1) * dq].astype(BF16)
                wuvbuf[n_hops] = wuv_ref[:, qq * dq:(qq + 1) * dq].astype(BF16)
                wqrstage[...] = wqr_ref[:, qq * HQ * Dr:(qq + 1) * HQ * Dr]
        xv = x_ref[...]
        cbuf[n_hops] = jnp.dot(
            xv, wdkv_ref[...], preferred_element_type=F32).astype(BF16)

        barrier = pltpu.get_barrier_semaphore()
        for nbr in (left, right):
            pl.semaphore_signal(
                barrier, inc=1,
                device_id=(mx, my, nbr),
                device_id_type=pl.DeviceIdType.MESH,
            )
        pl.semaphore_wait(barrier, 2)

        def start_hop(h, src_slot):
            rdmas = []
            for t, buf in enumerate((cbuf, wukbuf, wuvbuf)):
                rdma = pltpu.make_async_remote_copy(
                    src_ref=buf.at[src_slot],
                    dst_ref=buf.at[h],
                    send_sem=send_sems.at[h, t],
                    recv_sem=recv_sems.at[h, t],
                    device_id=(mx, my, right),
                    device_id_type=pl.DeviceIdType.MESH,
                )
                rdma.start()
                rdmas.append(rdma)
            return rdmas

        def fold(slot, first=False):
            cv = cbuf[slot]
            kp = jnp.dot(cv, wukbuf[slot], preferred_element_type=F32)
            vp = jnp.dot(cv, wuvbuf[slot], preferred_element_type=F32)
            if first:
                kacc[...] = kp
                vacc[...] = vp
            else:
                kacc[...] += kp
                vacc[...] += vp

        rdmas = start_hop(0, n_hops)
        fold(n_hops, first=True)
        qrq[...] = jnp.dot(xv, wqrstage[...], preferred_element_type=F32)
        krs[...] = jnp.dot(xv, wkr_ref[...], preferred_element_type=F32)
        wq_copy.wait()
        qs[:, 0:qchunk] = jnp.dot(
            xv, wqstage[:, 0:qchunk], preferred_element_type=F32)
        for r in rdmas:
            r.wait()

        for h in range(1, n_hops):
            rdmas = start_hop(h, h - 1)
            fold(h - 1)
            qs[:, h * qchunk:(h + 1) * qchunk] = jnp.dot(
                xv, wqstage[:, h * qchunk:(h + 1) * qchunk],
                preferred_element_type=F32)
            for r in rdmas:
                r.wait()

        fold(n_hops - 1)
        qs[:, n_hops * qchunk:] = jnp.dot(
            xv, wqstage[:, n_hops * qchunk:], preferred_element_type=F32)

        peers = []
        for i in (1, 2, 3):
            p_hq = hq ^ i
            peers.append((p_hq, p_hq // 2, p_hq % 2))
        for p_hq, px, py in peers:
            pl.semaphore_signal(
                xybar, inc=1,
                device_id=(px, py, mz),
                device_id_type=pl.DeviceIdType.MESH,
            )
        pl.semaphore_wait(xybar, 3)

        kr_v = krs[...]
        sends = []
        for j in range(HQ):
            q_h = qs[:, j * Dh:(j + 1) * Dh]
            k_h = kacc[:, j * Dh:(j + 1) * Dh]
            qr_h = qrq[:, j * Dr:(j + 1) * Dr]
            sc = lax.dot_general(
                q_h, k_h, (((1,), (1,)), ((), ())),
                preferred_element_type=F32,
            )
            sc += lax.dot_general(
                qr_h, kr_v, (((1,), (1,)), ((), ())),
                preferred_element_type=F32,
            )
            sc *= SCALE
            m = jnp.max(sc, axis=1, keepdims=True)
            p = jnp.exp(sc - m)
            p = p / jnp.sum(p, axis=1, keepdims=True)
            oq[:, j * Dh:(j + 1) * Dh] = jnp.dot(
                p, vacc[:, j * Dh:(j + 1) * Dh],
                preferred_element_type=F32).astype(BF16)
            for i, (p_hq, px, py) in enumerate(peers):
                rdma = pltpu.make_async_remote_copy(
                    src_ref=oq.at[:, pl.ds(j * Dh, Dh)],
                    dst_ref=oall.at[hq, :, pl.ds(j * Dh, Dh)],
                    send_sem=osend_sems.at[i, j],
                    recv_sem=orecv_sems.at[i, j],
                    device_id=(px, py, mz),
                    device_id_type=pl.DeviceIdType.MESH,
                )
                rdma.start()
                sends.append(rdma)

        wo_p1 = pltpu.make_async_copy(
            wo_ref.at[pl.ds(peers[0][0] * dq, dq), :], wo2.at[1],
            wo_sems.at[1])
        wo_p1.start()

        wo_own.wait()
        out_ref[...] = lax.dot_general(
            oq[...], wo2[0],
            (((1,), (0,)), ((), ())), preferred_element_type=F32)

        wo_next = wo_p1
        for i, (p_hq, px, py) in enumerate(peers):
            if i + 1 < len(peers):
                wo_after = pltpu.make_async_copy(
                    wo_ref.at[pl.ds(peers[i + 1][0] * dq, dq), :],
                    wo2.at[i % 2], wo_sems.at[i + 2])
                wo_after.start()
            for j in range(HQ):
                recv = pltpu.make_async_remote_copy(
                    src_ref=oq.at[:, pl.ds(j * Dh, Dh)],
                    dst_ref=oall.at[p_hq, :, pl.ds(j * Dh, Dh)],
                    send_sem=osend_sems.at[i, j],
                    recv_sem=orecv_sems.at[i, j],
                    device_id=(px, py, mz),
                    device_id_type=pl.DeviceIdType.MESH,
                )
                recv.wait_recv()
            wo_next.wait()
            out_ref[...] += lax.dot_general(
                oall[p_hq], wo2[(i + 1) % 2],
                (((1,), (0,)), ((), ())), preferred_element_type=F32)
            if i + 1 < len(peers):
                wo_next = wo_after
        for rdma in sends:
            rdma.wait_send()

    vm = pl.BlockSpec(memory_space=pltpu.VMEM)
    hbm = pl.BlockSpec(memory_space=pl.ANY)
    return pl.pallas_call(
        body,
        in_specs=[vm, vm, vm, vm, hbm, vm, vm, hbm],
        out_shape=jax.ShapeDtypeStruct((s, d), F32),
        scratch_shapes=[
            pltpu.VMEM((N_Z, s, DC), BF16),
            pltpu.VMEM((N_Z, DC, dq), BF16),
            pltpu.VMEM((N_Z, DC, dq), BF16),
            pltpu.VMEM((d, dq), F32),
            pltpu.VMEM((d, HQ * Dr), F32),
            pltpu.VMEM((s, dq), F32),
            pltpu.VMEM((s, dq), F32),
            pltpu.VMEM((s, dq), F32),
            pltpu.VMEM((s, HQ * Dr), F32),
            pltpu.VMEM((s, Dr), F32),
            pltpu.VMEM((s, dq), BF16),
            pltpu.VMEM((N_Q, s, dq), BF16),
            pltpu.VMEM((2, dq, d), F32),
            pltpu.SemaphoreType.DMA((n_hops, 3)),
            pltpu.SemaphoreType.DMA((n_hops, 3)),
            pltpu.SemaphoreType.DMA,
            pltpu.SemaphoreType.DMA((3, HQ)),
            pltpu.SemaphoreType.DMA((3, HQ)),
            pltpu.SemaphoreType.DMA((4,)),
            pltpu.SemaphoreType.REGULAR,
        ],
        compiler_params=pltpu.CompilerParams(
            collective_id=0, vmem_limit_bytes=63 * 1024 * 1024),
    )(x2, Wdkv, Wuk, Wuv, Wq, Wqr, Wkr, Wo)


def kernel(x, Wdkv, Wuk, Wuv, Wq, Wqr, Wkr, Wo):
    b, s, d = x.shape
    x2 = x.reshape(s, d)
    out = _mla_fused(x2, Wdkv, Wuk, Wuv, Wq, Wqr, Wkr, Wo)
    return out.reshape(b, s, d)


# device time: 84839 ns/iter; 2.4165x vs baseline; 1.0453x over previous
import jax
import jax.numpy as jnp
from jax import lax
from jax.experimental import pallas as pl
from jax.experimental.pallas import tpu as pltpu

N_Z = 4
N_Q = 4
H, Dh, Dr = 16, 128, 32
HQ = H // N_Q
DC = 128
SCALE = (Dh + Dr) ** -0.5
F32 = jnp.float32
BF16 = jnp.bfloat16


def _mla_fused(x3, Wdkv, Wuk, Wuv, Wq, Wqr, WkrT, Wo):
    _, s, d = x3.shape
    dq = d // N_Q
    n_hops = N_Z - 1
    qchunk = dq // N_Z

    def body(x_ref, wdkv_ref, wuk_ref, wuv_ref, wq_ref, wqr_ref, wkrt_ref,
             wo_ref, out_ref,
             cbuf, wukbuf, wuvbuf, wqstage, wqrstage,
             qs, kacc, vacc, qrq, krs, oq, oall, wo2,
             send_sems, recv_sems, wq_sem,
             osend_sems, orecv_sems, wo_sems, xybar):
        mx = lax.axis_index("x")
        my = lax.axis_index("y")
        mz = lax.axis_index("z")
        hq = mx * 2 + my
        left = (mz + N_Z - 1) % N_Z
        right = (mz + 1) % N_Z

        wq_copy = pltpu.make_async_copy(
            wq_ref.at[:, pl.ds(hq * dq, dq)], wqstage, wq_sem)
        wq_copy.start()
        wo_own = pltpu.make_async_copy(
            wo_ref.at[pl.ds(hq * dq, dq), :], wo2.at[0], wo_sems.at[0])
        wo_own.start()

        for qq in range(N_Q):
            @pl.when(hq == qq)
            def _():
                wukbuf[n_hops] = wuk_ref[:, qq * dq:(qq + 1) * dq].astype(BF16)
                wuvbuf[n_hops] = wuv_ref[:, qq * dq:(qq + 1) * dq].astype(BF16)
                wqrstage[...] = wqr_ref[:, qq * HQ * Dr:(qq + 1) * HQ * Dr]
        xv = x_ref[0]
        cbuf[n_hops] = jnp.dot(
            xv, wdkv_ref[...], preferred_element_type=F32).astype(BF16)

        barrier = pltpu.get_barrier_semaphore()
        for nbr in (left, right):
            pl.semaphore_signal(
                barrier, inc=1,
                device_id=(mx, my, nbr),
                device_id_type=pl.DeviceIdType.MESH,
            )
        pl.semaphore_wait(barrier, 2)

        def start_hop(h, src_slot):
            rdmas = []
            for t, buf in enumerate((cbuf, wukbuf, wuvbuf)):
                rdma = pltpu.make_async_remote_copy(
                    src_ref=buf.at[src_slot],
                    dst_ref=buf.at[h],
                    send_sem=send_sems.at[h, t],
                    recv_sem=recv_sems.at[h, t],
                    device_id=(mx, my, right),
                    device_id_type=pl.DeviceIdType.MESH,
                )
                rdma.start()
                rdmas.append(rdma)
            return rdmas

        def fold(slot, first=False):
            cv = cbuf[slot]
            kp = jnp.dot(cv, wukbuf[slot], preferred_element_type=F32)
            vp = jnp.dot(cv, wuvbuf[slot], preferred_element_type=F32)
            if first:
                kacc[...] = kp
                vacc[...] = vp
            else:
                kacc[...] += kp
                vacc[...] += vp

        rdmas = start_hop(0, n_hops)
        fold(n_hops, first=True)
        qrq[...] = jnp.dot(
            xv, wqrstage[...], preferred_element_type=F32) * SCALE
        krs[...] = lax.dot_general(
            xv, wkrt_ref[...], (((1,), (1,)), ((), ())),
            preferred_element_type=F32)
        wq_copy.wait()
        qs[:, 0:qchunk] = jnp.dot(
            xv, wqstage[:, 0:qchunk], preferred_element_type=F32) * SCALE
        for r in rdmas:
            r.wait()

        for h in range(1, n_hops):
            rdmas = start_hop(h, h - 1)
            fold(h - 1)
            qs[:, h * qchunk:(h + 1) * qchunk] = jnp.dot(
                xv, wqstage[:, h * qchunk:(h + 1) * qchunk],
                preferred_element_type=F32) * SCALE
            for r in rdmas:
                r.wait()

        fold(n_hops - 1)
        qs[:, n_hops * qchunk:] = jnp.dot(
            xv, wqstage[:, n_hops * qchunk:],
            preferred_element_type=F32) * SCALE

        peers = []
        for i in (1, 2, 3):
            p_hq = hq ^ i
            peers.append((p_hq, p_hq // 2, p_hq % 2))
        for p_hq, px, py in peers:
            pl.semaphore_signal(
                xybar, inc=1,
                device_id=(px, py, mz),
                device_id_type=pl.DeviceIdType.MESH,
            )
        pl.semaphore_wait(xybar, 3)

        kr_v = krs[...]
        sends = []
        for j in range(HQ):
            q_h = qs[:, j * Dh:(j + 1) * Dh]
            k_h = kacc[:, j * Dh:(j + 1) * Dh]
            qr_h = qrq[:, j * Dr:(j + 1) * Dr]
            sc = lax.dot_general(
                q_h, k_h, (((1,), (1,)), ((), ())),
                preferred_element_type=F32,
            )
            sc += lax.dot_general(
                qr_h, kr_v, (((1,), (1,)), ((), ())),
                preferred_element_type=F32,
            )
            p = jnp.exp(sc)
            denom = jnp.sum(p, axis=1, keepdims=True)
            o_un = jnp.dot(
                p, vacc[:, j * Dh:(j + 1) * Dh], preferred_element_type=F32)
            oq[:, j * Dh:(j + 1) * Dh] = (o_un / denom).astype(BF16)
            for i, (p_hq, px, py) in enumerate(peers):
                rdma = pltpu.make_async_remote_copy(
                    src_ref=oq.at[:, pl.ds(j * Dh, Dh)],
                    dst_ref=oall.at[hq, :, pl.ds(j * Dh, Dh)],
                    send_sem=osend_sems.at[i, j],
                    recv_sem=orecv_sems.at[i, j],
                    device_id=(px, py, mz),
                    device_id_type=pl.DeviceIdType.MESH,
                )
                rdma.start()
                sends.append(rdma)

        wo_p1 = pltpu.make_async_copy(
            wo_ref.at[pl.ds(peers[0][0] * dq, dq), :], wo2.at[1],
            wo_sems.at[1])
        wo_p1.start()

        wo_own.wait()
        out_ref[...] = lax.dot_general(
            oq[...], wo2[0],
            (((1,), (0,)), ((), ())), preferred_element_type=F32)

        wo_next = wo_p1
        for i, (p_hq, px, py) in enumerate(peers):
            if i + 1 < len(peers):
                wo_after = pltpu.make_async_copy(
                    wo_ref.at[pl.ds(peers[i + 1][0] * dq, dq), :],
                    wo2.at[i % 2], wo_sems.at[i + 2])
                wo_after.start()
            for j in range(HQ):
                recv = pltpu.make_async_remote_copy(
                    src_ref=oq.at[:, pl.ds(j * Dh, Dh)],
                    dst_ref=oall.at[p_hq, :, pl.ds(j * Dh, Dh)],
                    send_sem=osend_sems.at[i, j],
                    recv_sem=orecv_sems.at[i, j],
                    device_id=(px, py, mz),
                    device_id_type=pl.DeviceIdType.MESH,
                )
                recv.wait_recv()
            wo_next.wait()
            out_ref[...] += lax.dot_general(
                oall[p_hq], wo2[(i + 1) % 2],
                (((1,), (0,)), ((), ())), preferred_element_type=F32)
            if i + 1 < len(peers):
                wo_next = wo_after
        for rdma in sends:
            rdma.wait_send()

    vm = pl.BlockSpec(memory_space=pltpu.VMEM)
    hbm = pl.BlockSpec(memory_space=pl.ANY)
    return pl.pallas_call(
        body,
        in_specs=[vm, vm, vm, vm, hbm, vm, vm, hbm],
        out_shape=jax.ShapeDtypeStruct((s, d), F32),
        scratch_shapes=[
            pltpu.VMEM((N_Z, s, DC), BF16),
            pltpu.VMEM((N_Z, DC, dq), BF16),
            pltpu.VMEM((N_Z, DC, dq), BF16),
            pltpu.VMEM((d, dq), F32),
            pltpu.VMEM((d, HQ * Dr), F32),
            pltpu.VMEM((s, dq), F32),
            pltpu.VMEM((s, dq), F32),
            pltpu.VMEM((s, dq), F32),
            pltpu.VMEM((s, HQ * Dr), F32),
            pltpu.VMEM((s, Dr), F32),
            pltpu.VMEM((s, dq), BF16),
            pltpu.VMEM((N_Q, s, dq), BF16),
            pltpu.VMEM((2, dq, d), F32),
            pltpu.SemaphoreType.DMA((n_hops, 3)),
            pltpu.SemaphoreType.DMA((n_hops, 3)),
            pltpu.SemaphoreType.DMA,
            pltpu.SemaphoreType.DMA((3, HQ)),
            pltpu.SemaphoreType.DMA((3, HQ)),
            pltpu.SemaphoreType.DMA((4,)),
            pltpu.SemaphoreType.REGULAR,
        ],
        compiler_params=pltpu.CompilerParams(
            collective_id=0, vmem_limit_bytes=63 * 1024 * 1024),
    )(x3, Wdkv, Wuk, Wuv, Wq, Wqr, WkrT, Wo)


def kernel(x, Wdkv, Wuk, Wuv, Wq, Wqr, Wkr, Wo):
    b, s, d = x.shape
    out = _mla_fused(x, Wdkv, Wuk, Wuv, Wq, Wqr, Wkr.T, Wo)
    return out.reshape(b, s, d)


# device time: 84434 ns/iter; 2.4281x vs baseline; 1.0048x over previous
import jax
import jax.numpy as jnp
from jax import lax
from jax.experimental import pallas as pl
from jax.experimental.pallas import tpu as pltpu

N_Z = 4
N_Q = 4
H, Dh, Dr = 16, 128, 32
HQ = H // N_Q
DC = 128
SCALE = (Dh + Dr) ** -0.5
F32 = jnp.float32
BF16 = jnp.bfloat16


def _mla_fused(x3, Wdkv, Wuk, Wuv, Wq, Wqr, WkrT, Wo):
    _, s, d = x3.shape
    dq = d // N_Q
    n_hops = N_Z - 1
    qchunk = dq // N_Z

    def body(x_ref, wdkv_ref, wuk_ref, wuv_ref, wq_ref, wqr_ref, wkrt_ref,
             wo_ref, out_ref,
             cbuf, wukbuf, wuvbuf, wqstage, wqrstage,
             qs, kacc, vacc, qrq, krs, oq, oall, wo2,
             send_sems, recv_sems, wq_sem,
             osend_sems, orecv_sems, wo_sems, xybar):
        mx = lax.axis_index("x")
        my = lax.axis_index("y")
        mz = lax.axis_index("z")
        hq = mx * 2 + my
        left = (mz + N_Z - 1) % N_Z
        right = (mz + 1) % N_Z

        wq_copy = pltpu.make_async_copy(
            wq_ref.at[:, pl.ds(hq * dq, dq)], wqstage, wq_sem)
        wq_copy.start()
        wo_own = pltpu.make_async_copy(
            wo_ref.at[pl.ds(hq * dq, dq), :], wo2.at[0], wo_sems.at[0])
        wo_own.start()

        for qq in range(N_Q):
            @pl.when(hq == qq)
            def _():
                wukbuf[n_hops] = wuk_ref[:, qq * dq:(qq + 1) * dq].astype(BF16)
                wuvbuf[n_hops] = wuv_ref[:, qq * dq:(qq + 1) * dq].astype(BF16)
                wqrstage[...] = wqr_ref[:, qq * HQ * Dr:(qq + 1) * HQ * Dr]
        xv = x_ref[0]
        cbuf[n_hops] = jnp.dot(
            xv, wdkv_ref[...], preferred_element_type=F32).astype(BF16)

        barrier = pltpu.get_barrier_semaphore()
        for nbr in (left, right):
            pl.semaphore_signal(
                barrier, inc=1,
                device_id=(mx, my, nbr),
                device_id_type=pl.DeviceIdType.MESH,
            )
        pl.semaphore_wait(barrier, 2)

        def start_hop(h, src_slot):
            rdmas = []
            for t, buf in enumerate((cbuf, wukbuf, wuvbuf)):
                rdma = pltpu.make_async_remote_copy(
                    src_ref=buf.at[src_slot],
                    dst_ref=buf.at[h],
                    send_sem=send_sems.at[h, t],
                    recv_sem=recv_sems.at[h, t],
                    device_id=(mx, my, right),
                    device_id_type=pl.DeviceIdType.MESH,
                )
                rdma.start()
                rdmas.append(rdma)
            return rdmas

        def fold(slot, first=False):
            cv = cbuf[slot]
            kp = jnp.dot(cv, wukbuf[slot], preferred_element_type=F32)
            vp = jnp.dot(cv, wuvbuf[slot], preferred_element_type=F32)
            if first:
                kacc[...] = kp
                vacc[...] = vp
            else:
                kacc[...] += kp
                vacc[...] += vp

        rdmas = start_hop(0, n_hops)
        fold(n_hops, first=True)
        qrq[...] = jnp.dot(
            xv, wqrstage[...], preferred_element_type=F32) * SCALE
        krs[...] = lax.dot_general(
            xv, wkrt_ref[...], (((1,), (1,)), ((), ())),
            preferred_element_type=F32)
        wq_copy.wait()
        qs[:, 0:qchunk] = jnp.dot(
            xv, wqstage[:, 0:qchunk], preferred_element_type=F32) * SCALE
        for r in rdmas:
            r.wait()

        for h in range(1, n_hops):
            rdmas = start_hop(h, h - 1)
            fold(h - 1)
            qs[:, h * qchunk:(h + 1) * qchunk] = jnp.dot(
                xv, wqstage[:, h * qchunk:(h + 1) * qchunk],
                preferred_element_type=F32) * SCALE
            for r in rdmas:
                r.wait()

        fold(n_hops - 1)
        qs[:, n_hops * qchunk:] = jnp.dot(
            xv, wqstage[:, n_hops * qchunk:],
            preferred_element_type=F32) * SCALE

        peers = []
        for i in (1, 2, 3):
            p_hq = hq ^ i
            peers.append((p_hq, p_hq // 2, p_hq % 2))
        for p_hq, px, py in peers:
            pl.semaphore_signal(
                xybar, inc=1,
                device_id=(px, py, mz),
                device_id_type=pl.DeviceIdType.MESH,
            )
        pl.semaphore_wait(xybar, 3)

        kr_v = krs[...]
        sends = []
        for j in range(HQ):
            q_h = qs[:, j * Dh:(j + 1) * Dh]
            k_h = kacc[:, j * Dh:(j + 1) * Dh]
            qr_h = qrq[:, j * Dr:(j + 1) * Dr]
            sc = lax.dot_general(
                q_h, k_h, (((1,), (1,)), ((), ())),
                preferred_element_type=F32,
            )
            sc += lax.dot_general(
                qr_h, kr_v, (((1,), (1,)), ((), ())),
                preferred_element_type=F32,
            )
            p = jnp.exp(sc)
            denom = jnp.sum(p, axis=1, keepdims=True)
            o_un = jnp.dot(
                p, vacc[:, j * Dh:(j + 1) * Dh], preferred_element_type=F32)
            oq[:, j * Dh:(j + 1) * Dh] = (o_un / denom).astype(BF16)
            for i, (p_hq, px, py) in enumerate(peers):
                rdma = pltpu.make_async_remote_copy(
                    src_ref=oq.at[:, pl.ds(j * Dh, Dh)],
                    dst_ref=oall.at[hq, :, pl.ds(j * Dh, Dh)],
                    send_sem=osend_sems.at[i, j],
                    recv_sem=orecv_sems.at[i, j],
                    device_id=(px, py, mz),
                    device_id_type=pl.DeviceIdType.MESH,
                )
                rdma.start()
                sends.append(rdma)

        wo_p1 = pltpu.make_async_copy(
            wo_ref.at[pl.ds(peers[0][0] * dq, dq), :], wo2.at[1],
            wo_sems.at[1])
        wo_p1.start()

        wo_own.wait()
        out_ref[0] = lax.dot_general(
            oq[...], wo2[0],
            (((1,), (0,)), ((), ())), preferred_element_type=F32)

        wo_next = wo_p1
        for i, (p_hq, px, py) in enumerate(peers):
            if i + 1 < len(peers):
                wo_after = pltpu.make_async_copy(
                    wo_ref.at[pl.ds(peers[i + 1][0] * dq, dq), :],
                    wo2.at[i % 2], wo_sems.at[i + 2])
                wo_after.start()
            for j in range(HQ):
                recv = pltpu.make_async_remote_copy(
                    src_ref=oq.at[:, pl.ds(j * Dh, Dh)],
                    dst_ref=oall.at[p_hq, :, pl.ds(j * Dh, Dh)],
                    send_sem=osend_sems.at[i, j],
                    recv_sem=orecv_sems.at[i, j],
                    device_id=(px, py, mz),
                    device_id_type=pl.DeviceIdType.MESH,
                )
                recv.wait_recv()
            wo_next.wait()
            out_ref[0] += lax.dot_general(
                oall[p_hq], wo2[(i + 1) % 2],
                (((1,), (0,)), ((), ())), preferred_element_type=F32)
            if i + 1 < len(peers):
                wo_next = wo_after
        for rdma in sends:
            rdma.wait_send()

    vm = pl.BlockSpec(memory_space=pltpu.VMEM)
    hbm = pl.BlockSpec(memory_space=pl.ANY)
    return pl.pallas_call(
        body,
        in_specs=[vm, vm, vm, vm, hbm, vm, vm, hbm],
        out_shape=jax.ShapeDtypeStruct((1, s, d), F32),
        scratch_shapes=[
            pltpu.VMEM((N_Z, s, DC), BF16),
            pltpu.VMEM((N_Z, DC, dq), BF16),
            pltpu.VMEM((N_Z, DC, dq), BF16),
            pltpu.VMEM((d, dq), F32),
            pltpu.VMEM((d, HQ * Dr), F32),
            pltpu.VMEM((s, dq), F32),
            pltpu.VMEM((s, dq), F32),
            pltpu.VMEM((s, dq), F32),
            pltpu.VMEM((s, HQ * Dr), F32),
            pltpu.VMEM((s, Dr), F32),
            pltpu.VMEM((s, dq), BF16),
            pltpu.VMEM((N_Q, s, dq), BF16),
            pltpu.VMEM((2, dq, d), F32),
            pltpu.SemaphoreType.DMA((n_hops, 3)),
            pltpu.SemaphoreType.DMA((n_hops, 3)),
            pltpu.SemaphoreType.DMA,
            pltpu.SemaphoreType.DMA((3, HQ)),
            pltpu.SemaphoreType.DMA((3, HQ)),
            pltpu.SemaphoreType.DMA((4,)),
            pltpu.SemaphoreType.REGULAR,
        ],
        compiler_params=pltpu.CompilerParams(
            collective_id=0, vmem_limit_bytes=63 * 1024 * 1024),
    )(x3, Wdkv, Wuk, Wuv, Wq, Wqr, WkrT, Wo)


def kernel(x, Wdkv, Wuk, Wuv, Wq, Wqr, Wkr, Wo):
    return _mla_fused(x, Wdkv, Wuk, Wuv, Wq, Wqr, Wkr.T, Wo)


# device time: 82581 ns/iter; 2.4825x vs baseline; 1.0224x over previous
import jax
import jax.numpy as jnp
from jax import lax
from jax.experimental import pallas as pl
from jax.experimental.pallas import tpu as pltpu

N_Z = 4
N_Q = 4
H, Dh, Dr = 16, 128, 32
HQ = H // N_Q
DC = 128
SCALE = (Dh + Dr) ** -0.5
F32 = jnp.float32
BF16 = jnp.bfloat16


def _mla_fused(x3, Wdkv, Wuk, Wuv, Wq, Wqr, WkrT, Wo):
    _, s, d = x3.shape
    dq = d // N_Q
    n_hops = N_Z - 1
    qchunk = dq // N_Z

    def body(x_ref, wdkv_ref, wuk_ref, wuv_ref, wq_ref, wqr_ref, wkrt_ref,
             wo_ref, out_ref,
             cbuf, wukbuf, wuvbuf, wqstage, wqrstage,
             qs, kacc, vacc, qrq, krs, oq, oall, wo2,
             send_sems, recv_sems, wq_sem,
             osend_sems, orecv_sems, wo_sems, xybar):
        mx = lax.axis_index("x")
        my = lax.axis_index("y")
        mz = lax.axis_index("z")
        hq = mx * 2 + my
        left = (mz + N_Z - 1) % N_Z
        right = (mz + 1) % N_Z

        wq_copy = pltpu.make_async_copy(
            wq_ref.at[:, pl.ds(hq * dq, dq)], wqstage, wq_sem)
        wq_copy.start()
        wo_own = pltpu.make_async_copy(
            wo_ref.at[pl.ds(hq * dq, dq), :], wo2.at[0], wo_sems.at[0])
        wo_own.start()

        peers = []
        for i in (1, 2, 3):
            p_hq = hq ^ i
            peers.append((p_hq, p_hq // 2, p_hq % 2))
        for p_hq, px, py in peers:
            pl.semaphore_signal(
                xybar, inc=1,
                device_id=(px, py, mz),
                device_id_type=pl.DeviceIdType.MESH,
            )

        for qq in range(N_Q):
            @pl.when(hq == qq)
            def _():
                wukbuf[n_hops] = wuk_ref[:, qq * dq:(qq + 1) * dq].astype(BF16)
                wuvbuf[n_hops] = wuv_ref[:, qq * dq:(qq + 1) * dq].astype(BF16)
                wqrstage[...] = wqr_ref[:, qq * HQ * Dr:(qq + 1) * HQ * Dr]
        xv = x_ref[0]
        cbuf[n_hops] = jnp.dot(
            xv, wdkv_ref[...], preferred_element_type=F32).astype(BF16)

        barrier = pltpu.get_barrier_semaphore()
        for nbr in (left, right):
            pl.semaphore_signal(
                barrier, inc=1,
                device_id=(mx, my, nbr),
                device_id_type=pl.DeviceIdType.MESH,
            )
        pl.semaphore_wait(barrier, 2)

        def start_hop(h, src_slot):
            rdmas = []
            for t, buf in enumerate((cbuf, wukbuf, wuvbuf)):
                rdma = pltpu.make_async_remote_copy(
                    src_ref=buf.at[src_slot],
                    dst_ref=buf.at[h],
                    send_sem=send_sems.at[h, t],
                    recv_sem=recv_sems.at[h, t],
                    device_id=(mx, my, right),
                    device_id_type=pl.DeviceIdType.MESH,
                )
                rdma.start()
                rdmas.append(rdma)
            return rdmas

        def fold(slot, first=False):
            cv = cbuf[slot]
            kp = jnp.dot(cv, wukbuf[slot], preferred_element_type=F32)
            vp = jnp.dot(cv, wuvbuf[slot], preferred_element_type=F32)
            if first:
                kacc[...] = kp
                vacc[...] = vp
            else:
                kacc[...] += kp
                vacc[...] += vp

        rdmas = start_hop(0, n_hops)
        fold(n_hops, first=True)
        qrq[...] = jnp.dot(
            xv, wqrstage[...], preferred_element_type=F32) * SCALE
        krs[...] = lax.dot_general(
            xv, wkrt_ref[...], (((1,), (1,)), ((), ())),
            preferred_element_type=F32)
        wq_copy.wait()
        qs[:, 0:qchunk] = jnp.dot(
            xv, wqstage[:, 0:qchunk], preferred_element_type=F32) * SCALE
        for r in rdmas:
            r.wait()

        for h in range(1, n_hops):
            rdmas = start_hop(h, h - 1)
            fold(h - 1)
            qs[:, h * qchunk:(h + 1) * qchunk] = jnp.dot(
                xv, wqstage[:, h * qchunk:(h + 1) * qchunk],
                preferred_element_type=F32) * SCALE
            for r in rdmas:
                r.wait()

        fold(n_hops - 1)
        qs[:, n_hops * qchunk:] = jnp.dot(
            xv, wqstage[:, n_hops * qchunk:],
            preferred_element_type=F32) * SCALE

        pl.semaphore_wait(xybar, 3)

        kr_v = krs[...]
        sends = []
        for j in range(HQ):
            q_h = qs[:, j * Dh:(j + 1) * Dh]
            k_h = kacc[:, j * Dh:(j + 1) * Dh]
            qr_h = qrq[:, j * Dr:(j + 1) * Dr]
            sc = lax.dot_general(
                q_h, k_h, (((1,), (1,)), ((), ())),
                preferred_element_type=F32,
            )
            sc += lax.dot_general(
                qr_h, kr_v, (((1,), (1,)), ((), ())),
                preferred_element_type=F32,
            )
            p = jnp.exp(sc)
            denom = jnp.sum(p, axis=1, keepdims=True)
            o_un = jnp.dot(
                p, vacc[:, j * Dh:(j + 1) * Dh], preferred_element_type=F32)
            oq[:, j * Dh:(j + 1) * Dh] = (o_un / denom).astype(BF16)
            for i, (p_hq, px, py) in enumerate(peers):
                rdma = pltpu.make_async_remote_copy(
                    src_ref=oq.at[:, pl.ds(j * Dh, Dh)],
                    dst_ref=oall.at[hq, :, pl.ds(j * Dh, Dh)],
                    send_sem=osend_sems.at[i, j],
                    recv_sem=orecv_sems.at[i, j],
                    device_id=(px, py, mz),
                    device_id_type=pl.DeviceIdType.MESH,
                )
                rdma.start()
                sends.append(rdma)

        wo_p1 = pltpu.make_async_copy(
            wo_ref.at[pl.ds(peers[0][0] * dq, dq), :], wo2.at[1],
            wo_sems.at[1])
        wo_p1.start()

        wo_own.wait()
        out_ref[0] = lax.dot_general(
            oq[...], wo2[0],
            (((1,), (0,)), ((), ())), preferred_element_type=F32)

        wo_next = wo_p1
        for i, (p_hq, px, py) in enumerate(peers):
            if i + 1 < len(peers):
                wo_after = pltpu.make_async_copy(
                    wo_ref.at[pl.ds(peers[i + 1][0] * dq, dq), :],
                    wo2.at[i % 2], wo_sems.at[i + 2])
                wo_after.start()
            for j in range(HQ):
                recv = pltpu.make_async_remote_copy(
                    src_ref=oq.at[:, pl.ds(j * Dh, Dh)],
                    dst_ref=oall.at[p_hq, :, pl.ds(j * Dh, Dh)],
                    send_sem=osend_sems.at[i, j],
                    recv_sem=orecv_sems.at[i, j],
                    device_id=(px, py, mz),
                    device_id_type=pl.DeviceIdType.MESH,
                )
                recv.wait_recv()
            wo_next.wait()
            out_ref[0] += lax.dot_general(
                oall[p_hq], wo2[(i + 1) % 2],
                (((1,), (0,)), ((), ())), preferred_element_type=F32)
            if i + 1 < len(peers):
                wo_next = wo_after
        for rdma in sends:
            rdma.wait_send()

    vm = pl.BlockSpec(memory_space=pltpu.VMEM)
    hbm = pl.BlockSpec(memory_space=pl.ANY)
    return pl.pallas_call(
        body,
        in_specs=[vm, vm, vm, vm, hbm, vm, vm, hbm],
        out_shape=jax.ShapeDtypeStruct((1, s, d), F32),
        scratch_shapes=[
            pltpu.VMEM((N_Z, s, DC), BF16),
            pltpu.VMEM((N_Z, DC, dq), BF16),
            pltpu.VMEM((N_Z, DC, dq), BF16),
            pltpu.VMEM((d, dq), F32),
            pltpu.VMEM((d, HQ * Dr), F32),
            pltpu.VMEM((s, dq), F32),
            pltpu.VMEM((s, dq), F32),
            pltpu.VMEM((s, dq), F32),
            pltpu.VMEM((s, HQ * Dr), F32),
            pltpu.VMEM((s, Dr), F32),
            pltpu.VMEM((s, dq), BF16),
            pltpu.VMEM((N_Q, s, dq), BF16),
            pltpu.VMEM((2, dq, d), F32),
            pltpu.SemaphoreType.DMA((n_hops, 3)),
            pltpu.SemaphoreType.DMA((n_hops, 3)),
            pltpu.SemaphoreType.DMA,
            pltpu.SemaphoreType.DMA((3, HQ)),
            pltpu.SemaphoreType.DMA((3, HQ)),
            pltpu.SemaphoreType.DMA((4,)),
            pltpu.SemaphoreType.REGULAR,
        ],
        compiler_params=pltpu.CompilerParams(
            collective_id=0, vmem_limit_bytes=63 * 1024 * 1024),
    )(x3, Wdkv, Wuk, Wuv, Wq, Wqr, WkrT, Wo)


def kernel(x, Wdkv, Wuk, Wuv, Wq, Wqr, Wkr, Wo):
    return _mla_fused(x, Wdkv, Wuk, Wuv, Wq, Wqr, Wkr.T, Wo)
